# Initial kernel scaffold; baseline (speedup 1.0000x reference)
import jax
import jax.numpy as jnp
from jax.experimental import pallas as pl


def kernel(x, edge_index, edge_attr, batch, e_w0, e_b0, n_w0, n_b0, e_w1, e_b1, n_w1, n_b1, e_w2, e_b2, n_w2, n_b2):
    raise NotImplementedError("write your pallas kernel here")


# speedup vs baseline: 4.9921x; 4.9921x over previous
import functools

import jax
import jax.numpy as jnp
from jax import lax
from jax.experimental import pallas as pl
from jax.experimental.pallas import tpu as pltpu
from jax.experimental.pallas import tpu_sc as plsc

N = 10000
E = 320000
D = 128
DE = 16
G = 256

NC = 2
NS = 16
NW = NC * NS
EPW = E // NW
C = 80
HCH = 25
NG = EPW // (HCH * C)
NA = 10240
NPT = NA // NS


def _gine_sc_body(x_hbm, e_hbm, src_hbm, dst_hbm, acc_hbm,
                  src_v, dst_v, e_buf, x_buf, acc_sh,
                  sem_e0, sem_e1, sem_x0, sem_x1):
    c = lax.axis_index("c")
    s = lax.axis_index("s")
    w = c * NS + s
    sem_e = (sem_e0, sem_e1)
    sem_x = (sem_x0, sem_x1)

    z = jnp.zeros((16,), jnp.float32)

    def _zrow(j, carry):
        for k in range(8):
            e_buf[0, j, pl.ds(k * 16, 16)] = z
        return carry

    lax.fori_loop(0, C, _zrow, 0)
    row0 = s * NPT
    for r in range(NPT // C):
        pltpu.sync_copy(e_buf.at[0], acc_sh.at[pl.ds(row0 + r * C, C)])
    plsc.subcore_barrier()

    def _copies(g, j, b):
        eoff = g * (HCH * C) + j * C
        ec = pltpu.make_async_copy(
            e_hbm.at[pl.ds(eoff, C)], e_buf.at[b], sem_e[b])
        xc = pltpu.make_async_copy(
            x_hbm.at[src_v.at[j]], x_buf.at[b], sem_x[b])
        return ec, xc

    def _start(g, j, b):
        ec, xc = _copies(g, j, b)
        ec.start()
        xc.start()

    def _wait(g, j, b):
        ec, xc = _copies(g, j, b)
        ec.wait()
        xc.wait()

    def _compute(b):
        def _row(u, cc):
            for k in range(8):
                v = (e_buf[b, u, pl.ds(k * 16, 16)]
                     + x_buf[b, u, pl.ds(k * 16, 16)])
                e_buf[b, u, pl.ds(k * 16, 16)] = jnp.maximum(v, 0.0)
            return cc

        lax.fori_loop(0, C, _row, 0)

    for h in range(NG):
        g = w * NG + h
        pltpu.sync_copy(src_hbm.at[g], src_v)
        pltpu.sync_copy(dst_hbm.at[g], dst_v)

        _start(g, 0, 0)

        def _pair(p, carry):
            for b in range(2):
                j = p * 2 + b
                _start(g, j + 1, 1 - b)
                _wait(g, j, b)
                _compute(b)
                pltpu.sync_copy(e_buf.at[b], acc_sh.at[dst_v.at[j]],
                                add=True)
            return carry

        lax.fori_loop(0, HCH // 2, _pair, 0)
        last = HCH - 1
        _wait(g, last, 0)
        _compute(0)
        pltpu.sync_copy(e_buf.at[0], acc_sh.at[dst_v.at[last]], add=True)

    plsc.subcore_barrier()
    for r in range(NPT // C):
        rr = row0 + r * C
        pltpu.sync_copy(acc_sh.at[pl.ds(rr, C)], e_buf.at[0])
        pltpu.sync_copy(e_buf.at[0], acc_hbm.at[c, pl.ds(rr, C)])


@functools.cache
def _gine_sc_build():
    return functools.partial(
        pl.kernel,
        mesh=plsc.VectorSubcoreMesh(core_axis_name="c", subcore_axis_name="s"),
        out_type=jax.ShapeDtypeStruct((NC, NA, D), jnp.float32),
        scratch_types=[
            pltpu.VMEM((HCH, C), jnp.int32),
            pltpu.VMEM((HCH, C), jnp.int32),
            pltpu.VMEM((2, C, D), jnp.float32),
            pltpu.VMEM((2, C, D), jnp.float32),
            pltpu.VMEM_SHARED((NA, D), jnp.float32),
            pltpu.SemaphoreType.DMA,
            pltpu.SemaphoreType.DMA,
            pltpu.SemaphoreType.DMA,
            pltpu.SemaphoreType.DMA,
        ],
    )(_gine_sc_body)


BE = 4000


def _eproj_kernel(ea_ref, w_ref, b_ref, o0_ref, o1_ref, o2_ref):
    r = jnp.dot(ea_ref[...], w_ref[...],
                preferred_element_type=jnp.float32) + b_ref[...]
    o0_ref[...] = r[:, :D]
    o1_ref[...] = r[:, D:2 * D]
    o2_ref[...] = r[:, 2 * D:]


def _eproj(ea, w_all, b_all):
    return pl.pallas_call(
        _eproj_kernel,
        grid=(E // BE,),
        in_specs=[
            pl.BlockSpec((BE, DE), lambda i: (i, 0)),
            pl.BlockSpec((DE, 3 * D), lambda i: (0, 0)),
            pl.BlockSpec((1, 3 * D), lambda i: (0, 0)),
        ],
        out_specs=[pl.BlockSpec((BE, D), lambda i: (i, 0))] * 3,
        out_shape=[jax.ShapeDtypeStruct((E, D), jnp.float32)] * 3,
    )(ea, w_all, b_all)


BN = 1000


def _lfin_kernel(x_ref, a_ref, wt_ref, b_ref, o_ref):
    h = x_ref[...] + a_ref[0] + a_ref[1]
    o_ref[...] = jnp.maximum(
        jnp.dot(h, wt_ref[...], preferred_element_type=jnp.float32)
        + b_ref[...], 0.0)


def _lfin(x, acc, wt, b):
    return pl.pallas_call(
        _lfin_kernel,
        grid=(N // BN,),
        in_specs=[
            pl.BlockSpec((BN, D), lambda i: (i, 0)),
            pl.BlockSpec((NC, BN, D), lambda i: (0, i, 0)),
            pl.BlockSpec((D, D), lambda i: (0, 0)),
            pl.BlockSpec((1, D), lambda i: (0, 0)),
        ],
        out_specs=pl.BlockSpec((BN, D), lambda i: (i, 0)),
        out_shape=jax.ShapeDtypeStruct((N, D), jnp.float32),
    )(x, acc, wt, b)


NPAD = 10240
BP = 1024


def _pool_kernel(x_ref, batch_ref, o_ref, sums, counts):
    i = pl.program_id(0)

    @pl.when(i == 0)
    def _():
        sums[...] = jnp.zeros_like(sums)
        counts[...] = jnp.zeros_like(counts)

    bids = batch_ref[...]
    onehot = (bids == lax.broadcasted_iota(jnp.int32, (BP, G), 1)
              ).astype(jnp.float32)
    dn = (((0,), (0,)), ((), ()))
    sums[...] += lax.dot_general(onehot, x_ref[...], dn,
                                 preferred_element_type=jnp.float32)
    counts[...] += lax.dot_general(onehot, jnp.ones((BP, 1), jnp.float32), dn,
                                   preferred_element_type=jnp.float32)

    @pl.when(i == pl.num_programs(0) - 1)
    def _():
        o_ref[...] = sums[...] / jnp.maximum(counts[...], 1.0)


def _pool(xpad, bpad):
    return pl.pallas_call(
        _pool_kernel,
        grid=(NPAD // BP,),
        in_specs=[
            pl.BlockSpec((BP, D), lambda i: (i, 0)),
            pl.BlockSpec((BP, 1), lambda i: (i, 0)),
        ],
        out_specs=pl.BlockSpec((G, D), lambda i: (0, 0)),
        out_shape=jax.ShapeDtypeStruct((G, D), jnp.float32),
        scratch_shapes=[
            pltpu.VMEM((G, D), jnp.float32),
            pltpu.VMEM((G, 1), jnp.float32),
        ],
    )(xpad, bpad)


def kernel(x, edge_index, edge_attr, batch,
           e_w0, e_b0, n_w0, n_b0,
           e_w1, e_b1, n_w1, n_b1,
           e_w2, e_b2, n_w2, n_b2):
    x = x.astype(jnp.float32)
    edge_attr = edge_attr.astype(jnp.float32)
    src = edge_index[0].reshape(NW * NG, HCH, C)
    dst = edge_index[1].reshape(NW * NG, HCH, C)
    w_all = jnp.concatenate([e_w0.T, e_w1.T, e_w2.T], axis=1)
    b_all = jnp.concatenate([e_b0, e_b1, e_b2])[None, :]
    e0, e1, e2 = _eproj(edge_attr, w_all, b_all)
    xx = x
    for e, nw, nb in ((e0, n_w0, n_b0), (e1, n_w1, n_b1), (e2, n_w2, n_b2)):
        acc = _gine_sc_build()(xx, e, src, dst)
        xx = _lfin(xx, acc, nw.T, nb[None, :])
    xpad = jnp.concatenate([xx, jnp.zeros((NPAD - N, D), jnp.float32)], axis=0)
    bpad = jnp.concatenate(
        [batch, jnp.full((NPAD - N,), G, jnp.int32)], axis=0)[:, None]
    return _pool(xpad, bpad)
